# Initial kernel scaffold; baseline (speedup 1.0000x reference)
import jax
import jax.numpy as jnp
from jax.experimental import pallas as pl


def kernel(query_states, keys, values, priorities, Wq, bq, Wc, bc, ages, valid_mask):
    raise NotImplementedError("write your pallas kernel here")


# speedup vs baseline: 4.0897x; 4.0897x over previous
import functools

import jax
import jax.numpy as jnp
import numpy as np
from jax.experimental import pallas as pl
from jax.experimental.pallas import tpu as pltpu

_BUFFER_SIZE = 16384
_DECAY_RATE = 0.9
_ROUNDS = 10
_THRESHOLD = 0.5

_NB = 8
_BN = _BUFFER_SIZE // _NB
_BT = 256


def _body(qs_ref, keys_ref, values_ref, pri_ref, ages_ref, vm_ref,
          wq_ref, bq_ref, wc_ref, bc_ref, out_ref,
          qvec, s_scr, w_scr, acc):
    p = pl.program_id(0)
    j = pl.program_id(1)
    f32 = jnp.float32

    @pl.when(jnp.logical_and(p == 0, j == 0))
    def _init():
        qvec[...] = jnp.zeros_like(qvec)
        acc[...] = jnp.zeros_like(acc)

    @pl.when(p == 0)
    def _pool():
        qvec[...] += jnp.sum(qs_ref[...], axis=1)

    @pl.when(jnp.logical_and(p == 1, j == 0))
    def _project_q():
        q = qvec[...] * (1.0 / 2048.0)
        qvec[...] = jax.lax.dot_general(
            q, wq_ref[...], (((1,), (1,)), ((), ())),
            preferred_element_type=f32) + bq_ref[...]

    @pl.when(p == 1)
    def _scores():
        scale = 1.0 / np.sqrt(768.0).astype(np.float32)
        s_scr[j] = jax.lax.dot_general(
            qvec[...], keys_ref[...], (((1,), (1,)), ((), ())),
            preferred_element_type=f32) * scale

    @pl.when(jnp.logical_and(p == 2, j == 0))
    def _rounds():
        s = s_scr[...]
        log_decay = np.float32(np.log(_DECAY_RATE))
        eff0 = pri_ref[...] * jnp.exp(ages_ref[...] * log_decay)
        vm = vm_ref[...]
        wsum = jnp.zeros_like(s)
        for r in range(_ROUNDS):
            eff = eff0 * np.float32(_DECAY_RATE ** r)
            logits = s + jnp.log(eff + 1e-8)
            m = jnp.max(logits, axis=(0, 2), keepdims=True)
            pex = jnp.exp(logits - m)
            attn = pex / jnp.sum(pex, axis=(0, 2), keepdims=True)
            active = jax.nn.sigmoid((eff - _THRESHOLD) * 10.0) * vm
            a = attn * active
            wsum += a / (jnp.sum(a, axis=(0, 2), keepdims=True) + 1e-8)
        w_scr[...] = wsum

    @pl.when(p == 2)
    def _retrieve():
        acc[...] += jax.lax.dot_general(
            w_scr[j], values_ref[...], (((1,), (0,)), ((), ())),
            preferred_element_type=f32)

    @pl.when(jnp.logical_and(p == 2, j == _NB - 1))
    def _project_out():
        out_ref[...] = jax.lax.dot_general(
            acc[...], wc_ref[...], (((1,), (1,)), ((), ())),
            preferred_element_type=f32) * (1.0 / _ROUNDS) + bc_ref[...]


@jax.jit
def kernel(query_states, keys, values, priorities, Wq, bq, Wc, bc, ages,
           valid_mask):
    B, T, D = query_states.shape
    N = keys.shape[0]

    pri = priorities.reshape(_NB, 1, _BN)
    ages_f = ages.astype(jnp.float32).reshape(_NB, 1, _BN)
    vm = valid_mask.astype(jnp.float32).reshape(_NB, 1, _BN)
    bq2 = bq.reshape(1, D)
    bc2 = bc.reshape(1, D)

    n_t_blocks = T // _BT
    assert n_t_blocks == _NB

    grid = (3, _NB)
    out = pl.pallas_call(
        _body,
        grid=grid,
        in_specs=[
            pl.BlockSpec((B, _BT, D),
                         lambda p, j: (0, jnp.where(p == 0, j, _NB - 1), 0)),
            pl.BlockSpec((_BN, D),
                         lambda p, j: (jnp.where(p == 1, j,
                                                 jnp.where(p == 0, 0, _NB - 1)),
                                       0)),
            pl.BlockSpec((_BN, D), lambda p, j: (jnp.where(p == 2, j, 0), 0)),
            pl.BlockSpec((_NB, 1, _BN), lambda p, j: (0, 0, 0)),
            pl.BlockSpec((_NB, 1, _BN), lambda p, j: (0, 0, 0)),
            pl.BlockSpec((_NB, 1, _BN), lambda p, j: (0, 0, 0)),
            pl.BlockSpec((D, D), lambda p, j: (0, 0)),
            pl.BlockSpec((1, D), lambda p, j: (0, 0)),
            pl.BlockSpec((D, D), lambda p, j: (0, 0)),
            pl.BlockSpec((1, D), lambda p, j: (0, 0)),
        ],
        out_specs=pl.BlockSpec((B, D), lambda p, j: (0, 0)),
        out_shape=jax.ShapeDtypeStruct((B, D), jnp.float32),
        scratch_shapes=[
            pltpu.VMEM((B, D), jnp.float32),
            pltpu.VMEM((_NB, B, _BN), jnp.float32),
            pltpu.VMEM((_NB, B, _BN), jnp.float32),
            pltpu.VMEM((B, D), jnp.float32),
        ],
    )(query_states, keys, values, pri, ages_f, vm, Wq, bq2, Wc, bc2)
    return out
